# Initial kernel scaffold; baseline (speedup 1.0000x reference)
import jax
import jax.numpy as jnp
from jax.experimental import pallas as pl


def kernel(x, adj, W_lin, b_lin, W_eye, b_eye):
    raise NotImplementedError("write your pallas kernel here")


# speedup vs baseline: 2.8940x; 2.8940x over previous
import jax
import jax.numpy as jnp
from jax.experimental import pallas as pl

N = 2048
B = 2
IN = 64
CH = 64
HALF = CH // 2
R = B * CH
NOUT = N // 2

HT_BLK = 256
MP_JBLK = 256
MP_KBLK = 256


def _ht_kernel(adj_ref, x_ref, xb_ref, wl_ref, bl_ref, we_ref, be_ref, out_ref):
    adj_blk = adj_ref[...]
    for b in range(B):
        z = jax.lax.dot_general(
            x_ref[b], wl_ref[...],
            (((1,), (1,)), ((), ())),
            preferred_element_type=jnp.float32)
        lin = jax.lax.dot_general(
            adj_blk, z,
            (((1,), (0,)), ((), ())),
            preferred_element_type=jnp.float32)
        lin = lin + bl_ref[...][None, :]
        eye = jax.lax.dot_general(
            xb_ref[b], we_ref[...],
            (((1,), (1,)), ((), ())),
            preferred_element_type=jnp.float32)
        eye = eye + be_ref[...][None, :]
        out_ref[:, b * CH:b * CH + HALF] = lin
        out_ref[:, b * CH + HALF:(b + 1) * CH] = eye


def _maxpool_kernel(adj_ref, ht_ref, out_ref):
    @pl.when(pl.program_id(1) == 0)
    def _init():
        out_ref[...] = jnp.full((MP_JBLK, R), -jnp.inf, dtype=jnp.float32)

    a = adj_ref[...]
    h = ht_ref[...]
    acc = out_ref[...]
    for k in range(MP_KBLK):
        acc = jnp.maximum(acc, a[:, k:k + 1] * h[k:k + 1, :])
    out_ref[...] = acc


@jax.jit
def kernel(x, adj, W_lin, b_lin, W_eye, b_eye):
    hT = pl.pallas_call(
        _ht_kernel,
        grid=(N // HT_BLK,),
        in_specs=[
            pl.BlockSpec((HT_BLK, N), lambda i: (i, 0)),
            pl.BlockSpec((B, N, IN), lambda i: (0, 0, 0)),
            pl.BlockSpec((B, HT_BLK, IN), lambda i: (0, i, 0)),
            pl.BlockSpec((HALF, IN), lambda i: (0, 0)),
            pl.BlockSpec((HALF,), lambda i: (0,)),
            pl.BlockSpec((HALF, IN), lambda i: (0, 0)),
            pl.BlockSpec((HALF,), lambda i: (0,)),
        ],
        out_specs=pl.BlockSpec((HT_BLK, R), lambda i: (i, 0)),
        out_shape=jax.ShapeDtypeStruct((N, R), jnp.float32),
    )(adj, x, x, W_lin, b_lin, W_eye, b_eye)

    outT = pl.pallas_call(
        _maxpool_kernel,
        grid=(NOUT // MP_JBLK, N // MP_KBLK),
        in_specs=[
            pl.BlockSpec((MP_JBLK, MP_KBLK), lambda i, k: (i, k)),
            pl.BlockSpec((MP_KBLK, R), lambda i, k: (k, 0)),
        ],
        out_specs=pl.BlockSpec((MP_JBLK, R), lambda i, k: (i, 0)),
        out_shape=jax.ShapeDtypeStruct((NOUT, R), jnp.float32),
    )(adj, hT)

    return jnp.transpose(outT.reshape(NOUT, B, CH), (1, 0, 2))


# speedup vs baseline: 4.4147x; 1.5255x over previous
import functools

import jax
import jax.numpy as jnp
from jax import lax
from jax.experimental import pallas as pl
from jax.experimental.pallas import tpu as pltpu
from jax.experimental.pallas import tpu_sc as plsc

N = 2048
B = 2
IN = 64
CH = 64
HALF = CH // 2
R = B * CH
NOUT = N // 2

HT_BLK = 256

NTILES = 32
JPT = NOUT // NTILES
NQ = 4
KQ = N // NQ
L = 16
NCH = KQ // L


def _ht_kernel(adj_ref, x_ref, xb_ref, wl_ref, bl_ref, we_ref, be_ref, out_ref):
    adj_blk = adj_ref[...]
    for b in range(B):
        z = jax.lax.dot_general(
            x_ref[b], wl_ref[...],
            (((1,), (1,)), ((), ())),
            preferred_element_type=jnp.float32)
        lin = jax.lax.dot_general(
            adj_blk, z,
            (((1,), (0,)), ((), ())),
            preferred_element_type=jnp.float32)
        lin = lin + bl_ref[...][None, :]
        eye = jax.lax.dot_general(
            xb_ref[b], we_ref[...],
            (((1,), (1,)), ((), ())),
            preferred_element_type=jnp.float32)
        eye = eye + be_ref[...][None, :]
        out_ref[:, b * CH:b * CH + HALF] = lin
        out_ref[:, b * CH + HALF:(b + 1) * CH] = eye


def _compute_ht(x, adj):
    return pl.pallas_call(
        _ht_kernel,
        grid=(N // HT_BLK,),
        in_specs=[
            pl.BlockSpec((HT_BLK, N), lambda i: (i, 0)),
            pl.BlockSpec((B, N, IN), lambda i: (0, 0, 0)),
            pl.BlockSpec((B, HT_BLK, IN), lambda i: (0, i, 0)),
            pl.BlockSpec((HALF, IN), lambda i: (0, 0)),
            pl.BlockSpec((HALF,), lambda i: (0,)),
            pl.BlockSpec((HALF, IN), lambda i: (0, 0)),
            pl.BlockSpec((HALF,), lambda i: (0,)),
        ],
        out_specs=pl.BlockSpec((HT_BLK, R), lambda i: (i, 0)),
        out_shape=jax.ShapeDtypeStruct((N, R), jnp.float32),
    )


def _sc_maxpool_body(adj_hbm, ht_hbm, out_hbm, ht_v, arow_v, acc_v, zc_v):
    nc = 2
    wid = lax.axis_index("s") * nc + lax.axis_index("c")
    j0 = wid * JPT
    lanes = lax.iota(jnp.int32, L)
    neg_inf = jnp.full((L,), -jnp.inf, dtype=jnp.float32)
    zeros_i = jnp.zeros((L,), dtype=jnp.int32)

    for q in range(NQ):
        pltpu.sync_copy(ht_hbm.at[pl.ds(q * KQ * R, KQ * R)], ht_v)
        pltpu.sync_copy(adj_hbm.at[pl.ds(j0, JPT), pl.ds(q * KQ, KQ)], arow_v)

        def row_body(row, _, q=q):
            if q == 0:
                accs = [neg_inf for _ in range(R // L)]
                zc = zeros_i
            else:
                accs = [acc_v[pl.ds(row * R + g * L, L)] for g in range(R // L)]
                zc = zc_v[pl.ds(row * L, L)]

            def chunk_body(kc, carry):
                zc = carry[0]
                accs = list(carry[1:])
                av = arow_v[row, pl.ds(kc * L, L)]
                m = av != 0.0
                zc = zc + plsc.all_reduce_population_count(jnp.logical_not(m))

                def edge_cond(ec):
                    return jnp.any(ec[0])

                def edge_body(ec):
                    em = ec[0]
                    eaccs = list(ec[1:])
                    ffs = plsc.all_reduce_ffs(em)
                    col = kc * L + ffs
                    vsp = plsc.load_gather(
                        arow_v, [jnp.broadcast_to(row, (L,)).astype(jnp.int32), col])
                    kbase = col * R
                    new = []
                    for g in range(R // L):
                        hv = plsc.load_gather(ht_v, [kbase + (g * L) + lanes])
                        new.append(jnp.maximum(eaccs[g], vsp * hv))
                    em = jnp.logical_and(em, lanes != ffs)
                    return (em, *new)

                res = lax.while_loop(edge_cond, edge_body, (m, *accs))
                return (zc, *res[1:])

            out = lax.fori_loop(0, NCH, chunk_body, (zc, *accs))
            zc = out[0]
            accs = list(out[1:])

            if q == NQ - 1:
                hz = zc > 0
                accs = [jnp.where(hz, jnp.maximum(a, 0.0), a) for a in accs]
            for g in range(R // L):
                acc_v[pl.ds(row * R + g * L, L)] = accs[g]
            if q != NQ - 1:
                zc_v[pl.ds(row * L, L)] = zc
            return 0

        lax.fori_loop(0, JPT, row_body, 0)

    pltpu.sync_copy(acc_v, out_hbm.at[pl.ds(j0 * R, JPT * R)])


_sc_maxpool = functools.partial(
    pl.kernel,
    out_type=jax.ShapeDtypeStruct((NOUT * R,), jnp.float32),
    mesh=plsc.VectorSubcoreMesh(core_axis_name="c", subcore_axis_name="s"),
    scratch_types=[
        pltpu.VMEM((KQ * R,), jnp.float32),
        pltpu.VMEM((JPT, KQ), jnp.float32),
        pltpu.VMEM((JPT * R,), jnp.float32),
        pltpu.VMEM((JPT * L,), jnp.int32),
    ],
    compiler_params=pltpu.CompilerParams(needs_layout_passes=False),
)(_sc_maxpool_body)


@jax.jit
def kernel(x, adj, W_lin, b_lin, W_eye, b_eye):
    hT = _compute_ht(x, adj)(adj, x, x, W_lin, b_lin, W_eye, b_eye)
    outT = _sc_maxpool(adj, hT.reshape(-1)).reshape(NOUT, R)
    return jnp.transpose(outT.reshape(NOUT, B, CH), (1, 0, 2))


# speedup vs baseline: 6.2363x; 1.4126x over previous
import functools

import jax
import jax.numpy as jnp
from jax import lax
from jax.experimental import pallas as pl
from jax.experimental.pallas import tpu as pltpu
from jax.experimental.pallas import tpu_sc as plsc

N = 2048
B = 2
IN = 64
CH = 64
HALF = CH // 2
R = B * CH
NOUT = N // 2

HT_BLK = 256
CNT_BLK = 256

NTILES = 32
JPT = NOUT // NTILES
NQ = 4
KQ = N // NQ
L = 16
NCHUNK = N // L


def _ht_kernel(adj_ref, x_ref, xb_ref, wl_ref, bl_ref, we_ref, be_ref, out_ref):
    adj_blk = adj_ref[...]
    for b in range(B):
        z = jax.lax.dot_general(
            x_ref[b], wl_ref[...],
            (((1,), (1,)), ((), ())),
            preferred_element_type=jnp.float32)
        lin = jax.lax.dot_general(
            adj_blk, z,
            (((1,), (0,)), ((), ())),
            preferred_element_type=jnp.float32)
        lin = lin + bl_ref[...][None, :]
        eye = jax.lax.dot_general(
            xb_ref[b], we_ref[...],
            (((1,), (1,)), ((), ())),
            preferred_element_type=jnp.float32)
        eye = eye + be_ref[...][None, :]
        out_ref[:, b * CH:b * CH + HALF] = lin
        out_ref[:, b * CH + HALF:(b + 1) * CH] = eye


def _cnt_kernel(adj_ref, out_ref):
    nz = (adj_ref[...] != 0.0).astype(jnp.float32)
    kk = jax.lax.broadcasted_iota(jnp.int32, (N, NCHUNK), 0) // L
    cc = jax.lax.broadcasted_iota(jnp.int32, (N, NCHUNK), 1)
    sel = (kk == cc).astype(jnp.float32)
    cnt = jax.lax.dot_general(
        nz, sel, (((1,), (0,)), ((), ())),
        preferred_element_type=jnp.float32)
    out_ref[...] = cnt.astype(jnp.int32)


def _sc_maxpool_body(adj_hbm, ht_hbm, cnt_hbm, out_hbm,
                     ht_v, arow_v, acc_v, cnt_v):
    nc = 2
    wid = lax.axis_index("s") * nc + lax.axis_index("c")
    j0 = wid * JPT
    lanes = lax.iota(jnp.int32, L)
    neg_inf = jnp.full((L,), -jnp.inf, dtype=jnp.float32)
    nG = R // L

    pltpu.sync_copy(cnt_hbm.at[pl.ds(j0, JPT), :], cnt_v)

    for q in range(NQ):
        pltpu.sync_copy(ht_hbm.at[pl.ds(q * KQ * R, KQ * R)], ht_v)
        pltpu.sync_copy(adj_hbm.at[pl.ds(j0, JPT), pl.ds(q * KQ, KQ)], arow_v)

        def row_body(row, _, q=q):
            row_splat = jnp.broadcast_to(row, (L,)).astype(jnp.int32)
            if q == 0:
                accs = [neg_inf for _ in range(nG)]
            else:
                accs = [acc_v[pl.ds(row * R + g * L, L)] for g in range(nG)]

            for cg in range(KQ // L // L):
                cvec = cnt_v[row, pl.ds(q * (KQ // L) + cg * L, L)]
                om = cvec > 0

                def occ_cond(oc):
                    return jnp.any(oc[0])

                def occ_body(oc, cg=cg):
                    om = oc[0]
                    accs = list(oc[1:])
                    ffs_c = plsc.all_reduce_ffs(om)
                    lc = cg * L + ffs_c
                    av = plsc.load_gather(arow_v, [row_splat, lc * L + lanes])
                    m = av != 0.0

                    def edge_cond(ec):
                        return jnp.any(ec[0])

                    def edge_body(ec):
                        em = ec[0]
                        eaccs = list(ec[1:])
                        ffs = plsc.all_reduce_ffs(em)
                        col = lc * L + ffs
                        vsp = plsc.load_gather(arow_v, [row_splat, col])
                        kbase = col * R
                        new = []
                        for g in range(nG):
                            hv = plsc.load_gather(ht_v, [kbase + (g * L) + lanes])
                            new.append(jnp.maximum(eaccs[g], vsp * hv))
                        em = jnp.logical_and(em, lanes != ffs)
                        return (em, *new)

                    res = lax.while_loop(edge_cond, edge_body, (m, *accs))
                    om = jnp.logical_and(om, lanes != ffs_c)
                    return (om, *res[1:])

                out = lax.while_loop(occ_cond, occ_body, (om, *accs))
                accs = list(out[1:])

            if q == NQ - 1:
                nnz = jnp.zeros((L,), dtype=jnp.int32)
                for cg in range(NCHUNK // L):
                    nnz = nnz + cnt_v[row, pl.ds(cg * L, L)]
                tot = jnp.sum(nnz)
                hz = jnp.broadcast_to(tot < N, (L,))
                accs = [jnp.where(hz, jnp.maximum(a, 0.0), a) for a in accs]
            for g in range(nG):
                acc_v[pl.ds(row * R + g * L, L)] = accs[g]
            return 0

        lax.fori_loop(0, JPT, row_body, 0)

    pltpu.sync_copy(acc_v, out_hbm.at[pl.ds(j0 * R, JPT * R)])


_sc_maxpool = functools.partial(
    pl.kernel,
    out_type=jax.ShapeDtypeStruct((NOUT * R,), jnp.float32),
    mesh=plsc.VectorSubcoreMesh(core_axis_name="c", subcore_axis_name="s"),
    scratch_types=[
        pltpu.VMEM((KQ * R,), jnp.float32),
        pltpu.VMEM((JPT, KQ), jnp.float32),
        pltpu.VMEM((JPT * R,), jnp.float32),
        pltpu.VMEM((JPT, NCHUNK), jnp.int32),
    ],
    compiler_params=pltpu.CompilerParams(needs_layout_passes=False),
)(_sc_maxpool_body)


@jax.jit
def kernel(x, adj, W_lin, b_lin, W_eye, b_eye):
    hT = pl.pallas_call(
        _ht_kernel,
        grid=(N // HT_BLK,),
        in_specs=[
            pl.BlockSpec((HT_BLK, N), lambda i: (i, 0)),
            pl.BlockSpec((B, N, IN), lambda i: (0, 0, 0)),
            pl.BlockSpec((B, HT_BLK, IN), lambda i: (0, i, 0)),
            pl.BlockSpec((HALF, IN), lambda i: (0, 0)),
            pl.BlockSpec((HALF,), lambda i: (0,)),
            pl.BlockSpec((HALF, IN), lambda i: (0, 0)),
            pl.BlockSpec((HALF,), lambda i: (0,)),
        ],
        out_specs=pl.BlockSpec((HT_BLK, R), lambda i: (i, 0)),
        out_shape=jax.ShapeDtypeStruct((N, R), jnp.float32),
    )(adj, x, x, W_lin, b_lin, W_eye, b_eye)

    counts = pl.pallas_call(
        _cnt_kernel,
        grid=(NOUT // CNT_BLK,),
        in_specs=[pl.BlockSpec((CNT_BLK, N), lambda i: (i, 0))],
        out_specs=pl.BlockSpec((CNT_BLK, NCHUNK), lambda i: (i, 0)),
        out_shape=jax.ShapeDtypeStruct((NOUT, NCHUNK), jnp.int32),
    )(adj)

    outT = _sc_maxpool(adj, hT.reshape(-1), counts).reshape(NOUT, R)
    return jnp.transpose(outT.reshape(NOUT, B, CH), (1, 0, 2))
